# Initial kernel scaffold; baseline (speedup 1.0000x reference)
import jax
import jax.numpy as jnp
from jax import lax
from jax.experimental import pallas as pl
from jax.experimental.pallas import tpu as pltpu
from jax.experimental.pallas import tpu_sc as plsc

B = 4096
D = 64
T = 5
NO = 15
NI = 10
NC_TOT = 25
CB = 32
NCHUNK = B // CB
NWORK = 32
CPW = NCHUNK // NWORK

ROWS = CB + CB * NO + CB * NI
O_OFF = CB
I_OFF = CB + CB * NO
DOTS = CB * NC_TOT


def _sc_body(ti, to, wb, ob, ib, out, wix_v, oix_v, iix_v, acc, dotbuf, sem):
    nc = 2
    wid = lax.axis_index("s") * nc + lax.axis_index("c")

    def chunk_body(c, carry):
        g = wid * CPW + c
        pltpu.sync_copy(wb.at[g], wix_v)
        pltpu.sync_copy(ob.at[g], oix_v)
        pltpu.sync_copy(ib.at[g], iix_v)
        for t in range(T):
            add = t > 0
            cps = [pltpu.async_copy(ti.at[wix_v.at[t]], acc.at[pl.ds(0, CB)],
                                    sem, add=add)]
            for k in range(4):
                cps.append(pltpu.async_copy(
                    to.at[oix_v.at[t, k]],
                    acc.at[pl.ds(O_OFF + k * 120, 120)], sem, add=add))
            for k in range(4):
                cps.append(pltpu.async_copy(
                    ti.at[iix_v.at[t, k]],
                    acc.at[pl.ds(I_OFF + k * 80, 80)], sem, add=add))
            for cp in cps:
                cp.wait()

        def grp_body(grp, carry2):
            j = grp * 16 + lax.iota(jnp.int32, 16)
            b = j // NC_TOT
            rem = j - b * NC_TOT
            row = jnp.where(rem < NO,
                            O_OFF + b * NO + rem,
                            I_OFF + b * NI + (rem - NO))
            dot = jnp.zeros((16,), jnp.float32)
            for d in range(D):
                dcol = jnp.full((16,), d, jnp.int32)
                ctx = plsc.load_gather(acc, [row, dcol])
                inp = plsc.load_gather(acc, [b, dcol])
                dot = dot + ctx * inp
            dotbuf[pl.ds(grp * 16, 16)] = dot
            return carry2

        lax.fori_loop(0, DOTS // 16, grp_body, 0)
        pltpu.sync_copy(dotbuf, out.at[g])
        return carry

    lax.fori_loop(0, CPW, chunk_body, 0)


def _tc_body(dots_ref, ms_ref, ma_ref, out_ref):
    x = dots_ref[...]
    eps = jnp.float32(1e-10)
    col = lax.broadcasted_iota(jnp.int32, x.shape, 1)

    def sp(v):
        return jnp.maximum(v, 0.0) + jnp.log1p(jnp.exp(-jnp.abs(v)))

    sp_neg = sp(-(x + eps))
    sp_pos = sp(x - eps)
    ms = ms_ref[...]
    ma = ma_ref[...]
    zero = jnp.float32(0.0)
    p_s = jnp.sum(jnp.where(col < 5, sp_neg, zero))
    n_s = jnp.sum(jnp.where((col >= 5) & (col < 15), sp_pos, zero))
    s_s = jnp.sum(jnp.where((col >= 15) & (col < 20), sp_neg * ms, zero))
    a_s = jnp.sum(jnp.where(col >= 20, sp_pos * ma, zero))
    out_ref[0, 0] = p_s
    out_ref[0, 1] = n_s
    out_ref[0, 2] = s_s
    out_ref[0, 3] = a_s


def kernel(w_ix, p_ix, n_ix, s_ix, ms_ix, a_ix, ma_ix, table_i, table_o):
    w_blk = w_ix.reshape(NCHUNK, CB, T).transpose(0, 2, 1)
    o_cat = jnp.concatenate([p_ix, n_ix], axis=1)
    o_blk = (o_cat.reshape(NCHUNK, CB, NO, T).transpose(0, 3, 1, 2)
             .reshape(NCHUNK, T, 4, 120))
    i_cat = jnp.concatenate([s_ix, a_ix], axis=1)
    i_blk = (i_cat.reshape(NCHUNK, CB, NI, T).transpose(0, 3, 1, 2)
             .reshape(NCHUNK, T, 4, 80))

    mesh = plsc.VectorSubcoreMesh(core_axis_name="c", subcore_axis_name="s")
    sc = pl.kernel(
        _sc_body,
        out_type=jax.ShapeDtypeStruct((NCHUNK, DOTS), jnp.float32),
        mesh=mesh,
        scratch_types=[
            pltpu.VMEM((T, CB), jnp.int32),
            pltpu.VMEM((T, 4, 120), jnp.int32),
            pltpu.VMEM((T, 4, 80), jnp.int32),
            pltpu.VMEM((ROWS, D), jnp.float32),
            pltpu.VMEM((DOTS,), jnp.float32),
            pltpu.SemaphoreType.DMA,
        ],
    )
    dots = sc(table_i, table_o, w_blk, o_blk, i_blk).reshape(B, NC_TOT)

    scores = pl.pallas_call(
        _tc_body,
        out_shape=jax.ShapeDtypeStruct((1, 4), jnp.float32),
        out_specs=pl.BlockSpec(memory_space=pltpu.SMEM),
    )(dots, ms_ix, ma_ix)

    p_s = scores[0, 0] / B
    n_s = scores[0, 1] / B
    s_s = scores[0, 2] / B
    a_s = scores[0, 3] / B
    loss = p_s + n_s + s_s + a_s
    return (loss, p_s, n_s, s_s, a_s)


# speedup vs baseline: 4.3690x; 4.3690x over previous
import jax
import jax.numpy as jnp
import numpy as np
from jax import lax
from jax.experimental import pallas as pl
from jax.experimental.pallas import tpu as pltpu
from jax.experimental.pallas import tpu_sc as plsc

B = 4096
D = 64
T = 5
NO = 15
NI = 10
NC_TOT = 25
CB = 32
NCHUNK = B // CB
NWORK = 32
CPW = NCHUNK // NWORK

ROWS = CB + CB * NO + CB * NI
O_OFF = CB
I_OFF = CB + CB * NO
DOTS = CB * NC_TOT

_J = np.arange(DOTS)
_B = _J // NC_TOT
_REM = _J % NC_TOT
_ROWTAB = np.where(_REM < NO,
                   O_OFF + _B * NO + _REM,
                   I_OFF + _B * NI + (_REM - NO)).astype(np.int32)
_ROWTAB = _ROWTAB.reshape(DOTS // 16, 16)
_BTAB = _B.astype(np.int32).reshape(DOTS // 16, 16)


def _sc_body(ti, to, wb, ob, ib, rt, bt, out,
             wix_v, oix_v, iix_v, rt_v, bt_v, acc, dotbuf, sem):
    nc = 2
    wid = lax.axis_index("s") * nc + lax.axis_index("c")
    pltpu.sync_copy(rt, rt_v)
    pltpu.sync_copy(bt, bt_v)

    def chunk_body(c, carry):
        g = wid * CPW + c
        pltpu.sync_copy(wb.at[g], wix_v)
        pltpu.sync_copy(ob.at[g], oix_v)
        pltpu.sync_copy(ib.at[g], iix_v)
        for t in range(T):
            add = t > 0
            cps = [pltpu.async_copy(ti.at[wix_v.at[t]], acc.at[pl.ds(0, CB)],
                                    sem, add=add)]
            for k in range(4):
                cps.append(pltpu.async_copy(
                    to.at[oix_v.at[t, k]],
                    acc.at[pl.ds(O_OFF + k * 120, 120)], sem, add=add))
            for k in range(4):
                cps.append(pltpu.async_copy(
                    ti.at[iix_v.at[t, k]],
                    acc.at[pl.ds(I_OFF + k * 80, 80)], sem, add=add))
            for cp in cps:
                cp.wait()

        def grp_body(grp, carry2):
            row = rt_v[grp]
            b = bt_v[grp]
            dot = jnp.zeros((16,), jnp.float32)
            for d in range(D):
                dcol = jnp.full((16,), d, jnp.int32)
                ctx = plsc.load_gather(acc, [row, dcol])
                inp = plsc.load_gather(acc, [b, dcol])
                dot = dot + ctx * inp
            dotbuf[pl.ds(grp * 16, 16)] = dot
            return carry2

        lax.fori_loop(0, DOTS // 16, grp_body, 0)
        pltpu.sync_copy(dotbuf, out.at[g])
        return carry

    lax.fori_loop(0, CPW, chunk_body, 0)


def _tc_body(dots_ref, ms_ref, ma_ref, out_ref):
    x = dots_ref[...]
    eps = jnp.float32(1e-10)
    col = lax.broadcasted_iota(jnp.int32, x.shape, 1)

    def sp(v):
        return jnp.maximum(v, 0.0) + jnp.log1p(jnp.exp(-jnp.abs(v)))

    sp_neg = sp(-(x + eps))
    sp_pos = sp(x - eps)
    ms = ms_ref[...]
    ma = ma_ref[...]
    zero = jnp.float32(0.0)
    p_s = jnp.sum(jnp.where(col < 5, sp_neg, zero))
    n_s = jnp.sum(jnp.where((col >= 5) & (col < 15), sp_pos, zero))
    s_s = jnp.sum(jnp.where((col >= 15) & (col < 20), sp_neg * ms, zero))
    a_s = jnp.sum(jnp.where(col >= 20, sp_pos * ma, zero))
    out_ref[0, 0] = p_s
    out_ref[0, 1] = n_s
    out_ref[0, 2] = s_s
    out_ref[0, 3] = a_s


def kernel(w_ix, p_ix, n_ix, s_ix, ms_ix, a_ix, ma_ix, table_i, table_o):
    w_blk = w_ix.reshape(NCHUNK, CB, T).transpose(0, 2, 1)
    o_cat = jnp.concatenate([p_ix, n_ix], axis=1)
    o_blk = (o_cat.reshape(NCHUNK, CB, NO, T).transpose(0, 3, 1, 2)
             .reshape(NCHUNK, T, 4, 120))
    i_cat = jnp.concatenate([s_ix, a_ix], axis=1)
    i_blk = (i_cat.reshape(NCHUNK, CB, NI, T).transpose(0, 3, 1, 2)
             .reshape(NCHUNK, T, 4, 80))

    mesh = plsc.VectorSubcoreMesh(core_axis_name="c", subcore_axis_name="s")
    sc = pl.kernel(
        _sc_body,
        out_type=jax.ShapeDtypeStruct((NCHUNK, DOTS), jnp.float32),
        mesh=mesh,
        scratch_types=[
            pltpu.VMEM((T, CB), jnp.int32),
            pltpu.VMEM((T, 4, 120), jnp.int32),
            pltpu.VMEM((T, 4, 80), jnp.int32),
            pltpu.VMEM((DOTS // 16, 16), jnp.int32),
            pltpu.VMEM((DOTS // 16, 16), jnp.int32),
            pltpu.VMEM((ROWS, D), jnp.float32),
            pltpu.VMEM((DOTS,), jnp.float32),
            pltpu.SemaphoreType.DMA,
        ],
        compiler_params=pltpu.CompilerParams(use_tc_tiling_on_sc=False,
                                             needs_layout_passes=False),
    )
    dots = sc(table_i, table_o, w_blk, o_blk, i_blk,
              jnp.asarray(_ROWTAB), jnp.asarray(_BTAB)).reshape(B, NC_TOT)

    scores = pl.pallas_call(
        _tc_body,
        out_shape=jax.ShapeDtypeStruct((1, 4), jnp.float32),
        out_specs=pl.BlockSpec(memory_space=pltpu.SMEM),
    )(dots, ms_ix, ma_ix)

    p_s = scores[0, 0] / B
    n_s = scores[0, 1] / B
    s_s = scores[0, 2] / B
    a_s = scores[0, 3] / B
    loss = p_s + n_s + s_s + a_s
    return (loss, p_s, n_s, s_s, a_s)
